# baseline (speedup 1.0000x reference)
import functools

import jax
import jax.numpy as jnp
from jax import lax
from jax.experimental import pallas as pl
from jax.experimental.pallas import tpu as pltpu
from jax.experimental.pallas import tpu_sc as plsc

N_HE = 10000
EPS = 1e-5
_INTERP = False


def _pick(b, n):
    return b if n % b == 0 else n


def _mm_plain(act, W):
    n, fin = act.shape
    fout = W.shape[1]
    rb = _pick(1000, n)

    def body(a_ref, w_ref, o_ref):
        acc = jnp.dot(a_ref[...], w_ref[...], preferred_element_type=jnp.float32)
        hi = acc.astype(jnp.bfloat16)
        lo = (acc - hi.astype(jnp.float32)).astype(jnp.bfloat16)
        o_ref[...] = jnp.concatenate([hi, lo], axis=1)

    return pl.pallas_call(
        body,
        grid=(n // rb,),
        in_specs=[pl.BlockSpec((rb, fin), lambda i: (i, 0)),
                  pl.BlockSpec((fin, fout), lambda i: (0, 0))],
        out_specs=pl.BlockSpec((rb, 2 * fout), lambda i: (i, 0)),
        out_shape=jax.ShapeDtypeStruct((n, 2 * fout), jnp.bfloat16),
        interpret=_INTERP,
    )(act, W)


def _mm_bn(h, stats, g2d, bt2d, W):
    n, fin = h.shape
    fout = W.shape[1]
    rb = _pick(1000, n)
    inv_n = 1.0 / n

    def body(h_ref, s_ref, g_ref, b_ref, w_ref, o_ref):
        mean = s_ref[0:1, :] * inv_n
        var = s_ref[1:2, :] * inv_n - mean * mean
        scale = g_ref[...] * lax.rsqrt(var + EPS)
        shift = b_ref[...] - mean * scale
        a = h_ref[...] * scale + shift
        acc = jnp.dot(a, w_ref[...], preferred_element_type=jnp.float32)
        hi = acc.astype(jnp.bfloat16)
        lo = (acc - hi.astype(jnp.float32)).astype(jnp.bfloat16)
        o_ref[...] = jnp.concatenate([hi, lo], axis=1)

    return pl.pallas_call(
        body,
        grid=(n // rb,),
        in_specs=[pl.BlockSpec((rb, fin), lambda i: (i, 0)),
                  pl.BlockSpec((2, fin), lambda i: (0, 0)),
                  pl.BlockSpec((1, fin), lambda i: (0, 0)),
                  pl.BlockSpec((1, fin), lambda i: (0, 0)),
                  pl.BlockSpec((fin, fout), lambda i: (0, 0))],
        out_specs=pl.BlockSpec((rb, 2 * fout), lambda i: (i, 0)),
        out_shape=jax.ShapeDtypeStruct((n, 2 * fout), jnp.bfloat16),
        interpret=_INTERP,
    )(h, stats, g2d, bt2d, W)


def _ht_mm(H, xw, b_col):
    n, hep = H.shape
    f = xw.shape[1]
    cb = _pick(1024, hep)

    def body(h_ref, x_ref, s_ref, o_ref):
        acc = lax.dot_general(h_ref[...], x_ref[...],
                              (((0,), (0,)), ((), ())),
                              preferred_element_type=jnp.float32)
        fo = acc.shape[1] // 2
        cnt = s_ref[...]
        inv = jnp.where(cnt > 0, 1.0 / cnt, 0.0)
        val = (acc[:, :fo] + acc[:, fo:]) * inv
        hi = val.astype(jnp.bfloat16)
        lo = (val - hi.astype(jnp.float32)).astype(jnp.bfloat16)
        o_ref[...] = jnp.concatenate([hi, lo], axis=1)

    return pl.pallas_call(
        body,
        grid=(hep // cb,),
        in_specs=[pl.BlockSpec((n, cb), lambda i: (0, i)),
                  pl.BlockSpec((n, f), lambda i: (0, 0)),
                  pl.BlockSpec((cb, 1), lambda i: (i, 0))],
        out_specs=pl.BlockSpec((cb, f), lambda i: (i, 0)),
        out_shape=jax.ShapeDtypeStruct((hep, f), jnp.bfloat16),
        interpret=_INTERP,
    )(H, xw, b_col)


def _ht_mm_conv(H32, xw, b_col):
    n, hep = H32.shape
    f = xw.shape[1]
    cb = _pick(256, hep)

    def body(h_ref, x_ref, s_ref, o_ref, hb_ref):
        hb = h_ref[...].astype(jnp.bfloat16)
        hb_ref[...] = hb
        acc = lax.dot_general(hb, x_ref[...],
                              (((0,), (0,)), ((), ())),
                              preferred_element_type=jnp.float32)
        fo = acc.shape[1] // 2
        cnt = s_ref[...]
        inv = jnp.where(cnt > 0, 1.0 / cnt, 0.0)
        val = (acc[:, :fo] + acc[:, fo:]) * inv
        hi = val.astype(jnp.bfloat16)
        lo = (val - hi.astype(jnp.float32)).astype(jnp.bfloat16)
        o_ref[...] = jnp.concatenate([hi, lo], axis=1)

    return pl.pallas_call(
        body,
        grid=(hep // cb,),
        in_specs=[pl.BlockSpec((n, cb), lambda i: (0, i)),
                  pl.BlockSpec((n, f), lambda i: (0, 0)),
                  pl.BlockSpec((cb, 1), lambda i: (i, 0))],
        out_specs=[pl.BlockSpec((cb, f), lambda i: (i, 0)),
                   pl.BlockSpec((n, cb), lambda i: (0, i))],
        out_shape=[jax.ShapeDtypeStruct((hep, f), jnp.bfloat16),
                   jax.ShapeDtypeStruct((n, hep), jnp.bfloat16)],
        interpret=_INTERP,
    )(H32, xw, b_col)


def _h_mm(H, t, d_col, b2d):
    n, hep = H.shape
    f = t.shape[1]
    fo = f // 2
    rb = _pick(1000, n)

    def body(h_ref, t_ref, s_ref, b_ref, o_ref):
        acc = jnp.dot(h_ref[...], t_ref[...], preferred_element_type=jnp.float32)
        fo = acc.shape[1] // 2
        val = acc[:, :fo] + acc[:, fo:]
        cnt = s_ref[...]
        inv = jnp.where(cnt > 0, 1.0 / cnt, 0.0)
        o_ref[...] = jnp.maximum(val * inv + b_ref[...], 0.0)

    return pl.pallas_call(
        body,
        grid=(n // rb,),
        in_specs=[pl.BlockSpec((rb, hep), lambda i: (i, 0)),
                  pl.BlockSpec((hep, f), lambda i: (0, 0)),
                  pl.BlockSpec((rb, 1), lambda i: (i, 0)),
                  pl.BlockSpec((1, fo), lambda i: (0, 0))],
        out_specs=pl.BlockSpec((rb, fo), lambda i: (i, 0)),
        out_shape=jax.ShapeDtypeStruct((n, fo), jnp.float32),
        interpret=_INTERP,
    )(H, t, d_col, b2d)


def _bn_stats(h):
    n, f = h.shape
    rb = _pick(1000, n)

    def body(h_ref, o_ref):
        i = pl.program_id(0)

        @pl.when(i == 0)
        def _():
            o_ref[...] = jnp.zeros_like(o_ref)

        v = h_ref[...]
        s = jnp.sum(v, axis=0, keepdims=True)
        ss = jnp.sum(v * v, axis=0, keepdims=True)
        o_ref[...] += jnp.concatenate([s, ss], axis=0)

    return pl.pallas_call(
        body,
        grid=(n // rb,),
        in_specs=[pl.BlockSpec((rb, f), lambda i: (i, 0))],
        out_specs=pl.BlockSpec((2, f), lambda i: (0, 0)),
        out_shape=jax.ShapeDtypeStruct((2, f), jnp.float32),
        interpret=_INTERP,
    )(h)


_N = 10000
_HEP = 10240
_E = 320000
_EPT = 10240
_EP = 32 * _EPT
_K = 1_024_000
_KT = _K // 16
_TOTAL = _N * _HEP
_NPASS = _TOTAL // (2 * _K)
_SENT = 200000


def _build_incidence_sc(edge):
    mesh = plsc.VectorSubcoreMesh(core_axis_name="c", subcore_axis_name="s")
    pad = jnp.full((2, _EP - _E), _SENT, jnp.int32)
    edge_flat = jnp.concatenate([edge, pad], axis=1).reshape(2 * _EP)
    zer_b = jnp.zeros((_KT,), jnp.float32)

    @functools.partial(
        pl.kernel,
        out_type=(
            jax.ShapeDtypeStruct((_TOTAL,), jnp.float32),
            jax.ShapeDtypeStruct((_HEP,), jnp.float32),
            jax.ShapeDtypeStruct((_HEP,), jnp.float32),
        ),
        mesh=mesh,
        scratch_types=(
            pltpu.VMEM((_EPT,), jnp.int32),
            pltpu.VMEM((_EPT,), jnp.int32),
            pltpu.VMEM((_EPT,), jnp.int32),
            pltpu.VMEM((_EPT,), jnp.int32),
            pltpu.VMEM((80, 128), jnp.int32),
            pltpu.VMEM((80, 128), jnp.float32),
            pltpu.SemaphoreType.DMA,
            pltpu.VMEM_SHARED((_K + 1024,), jnp.float32),
            pltpu.VMEM_SHARED((_HEP + 1024,), jnp.float32),
        ),
    )
    def build(edge_ref, zer_ref, h_ref, d_ref, b_ref,
              n1, h1, n2, h2, idx2, vals2, sem, acc, accd):
        cid = lax.axis_index("c")
        sid = lax.axis_index("s")

        onef = jnp.ones((16,), jnp.float32)

        def _vf(i, _):
            vals2[i // 8, pl.ds((i % 8) * 16, 16)] = onef
            return 0
        lax.fori_loop(0, 640, _vf, 0, unroll=8)

        pltpu.sync_copy(edge_ref.at[pl.ds(pl.multiple_of(sid * _EPT, 256), _EPT)],
                        n1)
        pltpu.sync_copy(edge_ref.at[pl.ds(pl.multiple_of(_EP + sid * _EPT, 256), _EPT)],
                        h1)
        pltpu.sync_copy(edge_ref.at[pl.ds(pl.multiple_of((16 + sid) * _EPT, 256), _EPT)],
                        n2)
        pltpu.sync_copy(edge_ref.at[pl.ds(pl.multiple_of(_EP + (16 + sid) * _EPT, 256), _EPT)],
                        h2)

        pltpu.sync_copy(zer_ref, acc.at[pl.ds(pl.multiple_of(sid * _KT, 256), _KT)])

        @pl.when(sid == 0)
        def _():
            pltpu.sync_copy(zer_ref.at[pl.ds(0, 1024)], acc.at[pl.ds(_K, 1024)])
            pltpu.sync_copy(zer_ref.at[pl.ds(0, _HEP + 1024)], accd)
        plsc.subcore_barrier()

        def fill_idx(src, base, limit, trash):
            def _row(r, _):
                for c in range(8):
                    v = src[pl.ds(r * 128 + c * 16, 16)]
                    off = v - base
                    ok = (off >= 0) & (off < limit)
                    idx2[r, pl.ds(c * 16, 16)] = jnp.where(ok, off, trash + (v & 1023))
                return 0
            lax.fori_loop(0, 80, _row, 0)

        def scatter_add(dst):
            depth = 8

            def _s(j, _):
                @pl.when(j >= depth)
                def _():
                    pltpu.make_async_copy(vals2.at[0], dst.at[idx2.at[0]],
                                          sem).wait()
                pltpu.async_copy(vals2.at[j], dst.at[idx2.at[j]], sem, add=True)
                return 0
            lax.fori_loop(0, 80, _s, 0)

            def _d(j, _):
                pltpu.make_async_copy(vals2.at[0], dst.at[idx2.at[0]], sem).wait()
                return 0
            lax.fori_loop(0, depth, _d, 0)

        @pl.when(cid == 0)
        def _():
            fill_idx(n1, 0, _HEP, _HEP)
        @pl.when(cid == 1)
        def _():
            fill_idx(h1, 0, _HEP, _HEP)
        scatter_add(accd)
        @pl.when(cid == 0)
        def _():
            fill_idx(n2, 0, _HEP, _HEP)
        @pl.when(cid == 1)
        def _():
            fill_idx(h2, 0, _HEP, _HEP)
        scatter_add(accd)
        plsc.subcore_barrier()

        @pl.when((sid == 0) & (cid == 0))
        def _():
            pltpu.sync_copy(accd.at[pl.ds(0, _HEP)], d_ref)

        @pl.when((sid == 0) & (cid == 1))
        def _():
            pltpu.sync_copy(accd.at[pl.ds(0, _HEP)], b_ref)

        def _keys(r, _):
            for c in range(8):
                s = r * 128 + c * 16
                n1[pl.ds(s, 16)] = n1[pl.ds(s, 16)] * _HEP + h1[pl.ds(s, 16)]
                n2[pl.ds(s, 16)] = n2[pl.ds(s, 16)] * _HEP + h2[pl.ds(s, 16)]
            return 0
        lax.fori_loop(0, 80, _keys, 0)

        def _pass(p, _):
            base = (2 * p + cid) * _K
            fill_idx(n1, base, _K, _K)
            scatter_add(acc)
            fill_idx(n2, base, _K, _K)
            scatter_add(acc)
            plsc.subcore_barrier()
            off = sid * _KT
            pltpu.sync_copy(acc.at[pl.ds(pl.multiple_of(off, 256), _KT)],
                            h_ref.at[pl.ds(pl.multiple_of(base + off, 256), _KT)])
            pltpu.sync_copy(zer_ref, acc.at[pl.ds(pl.multiple_of(off, 256), _KT)])
            plsc.subcore_barrier()
            return 0
        lax.fori_loop(0, _NPASS, _pass, 0)

    return build(edge_flat, zer_b)


def _encode(x, H32, dcol, bcol, W1, b1, g1, bt1, W2, b2, g2, bt2, W3, b3):
    H = None
    h = x
    for (W, b, g, bt) in ((W1, b1, None, None),
                          (W2, b2, g1, bt1),
                          (W3, b3, g2, bt2)):
        if g is None:
            xw = _mm_plain(h, W)
            t, H = _ht_mm_conv(H32, xw, bcol)
        else:
            stats = _bn_stats(h)
            xw = _mm_bn(h, stats, g.reshape(1, -1), bt.reshape(1, -1), W)
            t = _ht_mm(H, xw, bcol)
        h = _h_mm(H, t, dcol, b.reshape(1, -1))
    return h


def kernel(x, edge, W1, b1, g1, bt1, W2, b2, g2, bt2, W3, b3):
    H_flat, D, B = _build_incidence_sc(edge)
    return _encode(x, H_flat.reshape(_N, _HEP), D[:_N].reshape(_N, 1),
                   B.reshape(_HEP, 1), W1, b1, g1, bt1, W2, b2, g2, bt2, W3, b3)


# speedup vs baseline: 1.0098x; 1.0098x over previous
import functools

import jax
import jax.numpy as jnp
from jax import lax
from jax.experimental import pallas as pl
from jax.experimental.pallas import tpu as pltpu
from jax.experimental.pallas import tpu_sc as plsc

N_HE = 10000
EPS = 1e-5
_INTERP = False


def _pick(b, n):
    return b if n % b == 0 else n


def _mm_plain(act, W):
    n, fin = act.shape
    fout = W.shape[1]
    rb = _pick(1000, n)

    def body(a_ref, w_ref, o_ref):
        acc = jnp.dot(a_ref[...], w_ref[...], preferred_element_type=jnp.float32)
        hi = acc.astype(jnp.bfloat16)
        lo = (acc - hi.astype(jnp.float32)).astype(jnp.bfloat16)
        o_ref[...] = jnp.concatenate([hi, lo], axis=1)

    return pl.pallas_call(
        body,
        grid=(n // rb,),
        in_specs=[pl.BlockSpec((rb, fin), lambda i: (i, 0)),
                  pl.BlockSpec((fin, fout), lambda i: (0, 0))],
        out_specs=pl.BlockSpec((rb, 2 * fout), lambda i: (i, 0)),
        out_shape=jax.ShapeDtypeStruct((n, 2 * fout), jnp.bfloat16),
        interpret=_INTERP,
    )(act, W)


def _mm_bn(h, stats, g2d, bt2d, W):
    n, fin = h.shape
    fout = W.shape[1]
    rb = _pick(1000, n)
    inv_n = 1.0 / n

    def body(h_ref, s_ref, g_ref, b_ref, w_ref, o_ref):
        mean = s_ref[0:1, :] * inv_n
        var = s_ref[1:2, :] * inv_n - mean * mean
        scale = g_ref[...] * lax.rsqrt(var + EPS)
        shift = b_ref[...] - mean * scale
        a = h_ref[...] * scale + shift
        acc = jnp.dot(a, w_ref[...], preferred_element_type=jnp.float32)
        hi = acc.astype(jnp.bfloat16)
        lo = (acc - hi.astype(jnp.float32)).astype(jnp.bfloat16)
        o_ref[...] = jnp.concatenate([hi, lo], axis=1)

    return pl.pallas_call(
        body,
        grid=(n // rb,),
        in_specs=[pl.BlockSpec((rb, fin), lambda i: (i, 0)),
                  pl.BlockSpec((2, fin), lambda i: (0, 0)),
                  pl.BlockSpec((1, fin), lambda i: (0, 0)),
                  pl.BlockSpec((1, fin), lambda i: (0, 0)),
                  pl.BlockSpec((fin, fout), lambda i: (0, 0))],
        out_specs=pl.BlockSpec((rb, 2 * fout), lambda i: (i, 0)),
        out_shape=jax.ShapeDtypeStruct((n, 2 * fout), jnp.bfloat16),
        interpret=_INTERP,
    )(h, stats, g2d, bt2d, W)


def _ht_mm(H, xw, b_col):
    n, hep = H.shape
    f = xw.shape[1]
    cb = _pick(512, hep)

    def body(h_ref, x_ref, s_ref, o_ref):
        acc = lax.dot_general(h_ref[...], x_ref[...],
                              (((0,), (0,)), ((), ())),
                              preferred_element_type=jnp.float32)
        fo = acc.shape[1] // 2
        cnt = s_ref[...]
        inv = jnp.where(cnt > 0, 1.0 / cnt, 0.0)
        val = (acc[:, :fo] + acc[:, fo:]) * inv
        hi = val.astype(jnp.bfloat16)
        lo = (val - hi.astype(jnp.float32)).astype(jnp.bfloat16)
        o_ref[...] = jnp.concatenate([hi, lo], axis=1)

    return pl.pallas_call(
        body,
        grid=(hep // cb,),
        in_specs=[pl.BlockSpec((n, cb), lambda i: (0, i)),
                  pl.BlockSpec((n, f), lambda i: (0, 0)),
                  pl.BlockSpec((cb, 1), lambda i: (i, 0))],
        out_specs=pl.BlockSpec((cb, f), lambda i: (i, 0)),
        out_shape=jax.ShapeDtypeStruct((hep, f), jnp.bfloat16),
        interpret=_INTERP,
    )(H, xw, b_col)


def _ht_mm_conv(H32, xw, b_col):
    n, hep = H32.shape
    f = xw.shape[1]
    cb = _pick(256, hep)

    def body(h_ref, x_ref, s_ref, o_ref, hb_ref):
        hb = h_ref[...].astype(jnp.bfloat16)
        hb_ref[...] = hb
        acc = lax.dot_general(hb, x_ref[...],
                              (((0,), (0,)), ((), ())),
                              preferred_element_type=jnp.float32)
        fo = acc.shape[1] // 2
        cnt = s_ref[...]
        inv = jnp.where(cnt > 0, 1.0 / cnt, 0.0)
        val = (acc[:, :fo] + acc[:, fo:]) * inv
        hi = val.astype(jnp.bfloat16)
        lo = (val - hi.astype(jnp.float32)).astype(jnp.bfloat16)
        o_ref[...] = jnp.concatenate([hi, lo], axis=1)

    return pl.pallas_call(
        body,
        grid=(hep // cb,),
        in_specs=[pl.BlockSpec((n, cb), lambda i: (0, i)),
                  pl.BlockSpec((n, f), lambda i: (0, 0)),
                  pl.BlockSpec((cb, 1), lambda i: (i, 0))],
        out_specs=[pl.BlockSpec((cb, f), lambda i: (i, 0)),
                   pl.BlockSpec((n, cb), lambda i: (0, i))],
        out_shape=[jax.ShapeDtypeStruct((hep, f), jnp.bfloat16),
                   jax.ShapeDtypeStruct((n, hep), jnp.bfloat16)],
        interpret=_INTERP,
    )(H32, xw, b_col)


def _h_mm(H, t, d_col, b2d):
    n, hep = H.shape
    f = t.shape[1]
    fo = f // 2
    rb = _pick(400, n)

    def body(h_ref, t_ref, s_ref, b_ref, o_ref):
        acc = jnp.dot(h_ref[...], t_ref[...], preferred_element_type=jnp.float32)
        fo = acc.shape[1] // 2
        val = acc[:, :fo] + acc[:, fo:]
        cnt = s_ref[...]
        inv = jnp.where(cnt > 0, 1.0 / cnt, 0.0)
        o_ref[...] = jnp.maximum(val * inv + b_ref[...], 0.0)

    return pl.pallas_call(
        body,
        grid=(n // rb,),
        in_specs=[pl.BlockSpec((rb, hep), lambda i: (i, 0)),
                  pl.BlockSpec((hep, f), lambda i: (0, 0)),
                  pl.BlockSpec((rb, 1), lambda i: (i, 0)),
                  pl.BlockSpec((1, fo), lambda i: (0, 0))],
        out_specs=pl.BlockSpec((rb, fo), lambda i: (i, 0)),
        out_shape=jax.ShapeDtypeStruct((n, fo), jnp.float32),
        interpret=_INTERP,
    )(H, t, d_col, b2d)


def _bn_stats(h):
    n, f = h.shape
    rb = _pick(1000, n)

    def body(h_ref, o_ref):
        i = pl.program_id(0)

        @pl.when(i == 0)
        def _():
            o_ref[...] = jnp.zeros_like(o_ref)

        v = h_ref[...]
        s = jnp.sum(v, axis=0, keepdims=True)
        ss = jnp.sum(v * v, axis=0, keepdims=True)
        o_ref[...] += jnp.concatenate([s, ss], axis=0)

    return pl.pallas_call(
        body,
        grid=(n // rb,),
        in_specs=[pl.BlockSpec((rb, f), lambda i: (i, 0))],
        out_specs=pl.BlockSpec((2, f), lambda i: (0, 0)),
        out_shape=jax.ShapeDtypeStruct((2, f), jnp.float32),
        interpret=_INTERP,
    )(h)


_N = 10000
_HEP = 10240
_E = 320000
_EPT = 10240
_EP = 32 * _EPT
_K = 1_024_000
_KT = _K // 16
_TOTAL = _N * _HEP
_NPASS = _TOTAL // (2 * _K)
_SENT = 200000


def _build_incidence_sc(edge):
    mesh = plsc.VectorSubcoreMesh(core_axis_name="c", subcore_axis_name="s")
    pad = jnp.full((2, _EP - _E), _SENT, jnp.int32)
    edge_flat = jnp.concatenate([edge, pad], axis=1).reshape(2 * _EP)
    zer_b = jnp.zeros((_KT,), jnp.float32)

    @functools.partial(
        pl.kernel,
        out_type=(
            jax.ShapeDtypeStruct((_TOTAL,), jnp.float32),
            jax.ShapeDtypeStruct((_HEP,), jnp.float32),
            jax.ShapeDtypeStruct((_HEP,), jnp.float32),
        ),
        mesh=mesh,
        scratch_types=(
            pltpu.VMEM((_EPT,), jnp.int32),
            pltpu.VMEM((_EPT,), jnp.int32),
            pltpu.VMEM((_EPT,), jnp.int32),
            pltpu.VMEM((_EPT,), jnp.int32),
            pltpu.VMEM((80, 128), jnp.int32),
            pltpu.VMEM((80, 128), jnp.float32),
            pltpu.SemaphoreType.DMA,
            pltpu.VMEM_SHARED((_K + 1024,), jnp.float32),
            pltpu.VMEM_SHARED((_HEP + 1024,), jnp.float32),
        ),
    )
    def build(edge_ref, zer_ref, h_ref, d_ref, b_ref,
              n1, h1, n2, h2, idx2, vals2, sem, acc, accd):
        cid = lax.axis_index("c")
        sid = lax.axis_index("s")

        onef = jnp.ones((16,), jnp.float32)

        def _vf(i, _):
            vals2[i // 8, pl.ds((i % 8) * 16, 16)] = onef
            return 0
        lax.fori_loop(0, 640, _vf, 0, unroll=8)

        pltpu.sync_copy(edge_ref.at[pl.ds(pl.multiple_of(sid * _EPT, 256), _EPT)],
                        n1)
        pltpu.sync_copy(edge_ref.at[pl.ds(pl.multiple_of(_EP + sid * _EPT, 256), _EPT)],
                        h1)
        pltpu.sync_copy(edge_ref.at[pl.ds(pl.multiple_of((16 + sid) * _EPT, 256), _EPT)],
                        n2)
        pltpu.sync_copy(edge_ref.at[pl.ds(pl.multiple_of(_EP + (16 + sid) * _EPT, 256), _EPT)],
                        h2)

        pltpu.sync_copy(zer_ref, acc.at[pl.ds(pl.multiple_of(sid * _KT, 256), _KT)])

        @pl.when(sid == 0)
        def _():
            pltpu.sync_copy(zer_ref.at[pl.ds(0, 1024)], acc.at[pl.ds(_K, 1024)])
            pltpu.sync_copy(zer_ref.at[pl.ds(0, _HEP + 1024)], accd)
        plsc.subcore_barrier()

        def fill_idx(src, base, limit, trash):
            def _row(r, _):
                for c in range(8):
                    v = src[pl.ds(r * 128 + c * 16, 16)]
                    off = v - base
                    ok = (off >= 0) & (off < limit)
                    idx2[r, pl.ds(c * 16, 16)] = jnp.where(ok, off, trash + (v & 1023))
                return 0
            lax.fori_loop(0, 80, _row, 0)

        def scatter_add(dst):
            depth = 8

            def _s(j, _):
                @pl.when(j >= depth)
                def _():
                    pltpu.make_async_copy(vals2.at[0], dst.at[idx2.at[0]],
                                          sem).wait()
                pltpu.async_copy(vals2.at[j], dst.at[idx2.at[j]], sem, add=True)
                return 0
            lax.fori_loop(0, 80, _s, 0)

            def _d(j, _):
                pltpu.make_async_copy(vals2.at[0], dst.at[idx2.at[0]], sem).wait()
                return 0
            lax.fori_loop(0, depth, _d, 0)

        @pl.when(cid == 0)
        def _():
            fill_idx(n1, 0, _HEP, _HEP)
        @pl.when(cid == 1)
        def _():
            fill_idx(h1, 0, _HEP, _HEP)
        scatter_add(accd)
        @pl.when(cid == 0)
        def _():
            fill_idx(n2, 0, _HEP, _HEP)
        @pl.when(cid == 1)
        def _():
            fill_idx(h2, 0, _HEP, _HEP)
        scatter_add(accd)
        plsc.subcore_barrier()

        @pl.when((sid == 0) & (cid == 0))
        def _():
            pltpu.sync_copy(accd.at[pl.ds(0, _HEP)], d_ref)

        @pl.when((sid == 0) & (cid == 1))
        def _():
            pltpu.sync_copy(accd.at[pl.ds(0, _HEP)], b_ref)

        def _keys(r, _):
            for c in range(8):
                s = r * 128 + c * 16
                n1[pl.ds(s, 16)] = n1[pl.ds(s, 16)] * _HEP + h1[pl.ds(s, 16)]
                n2[pl.ds(s, 16)] = n2[pl.ds(s, 16)] * _HEP + h2[pl.ds(s, 16)]
            return 0
        lax.fori_loop(0, 80, _keys, 0)

        def _pass(p, _):
            base = (2 * p + cid) * _K
            fill_idx(n1, base, _K, _K)
            scatter_add(acc)
            fill_idx(n2, base, _K, _K)
            scatter_add(acc)
            plsc.subcore_barrier()
            off = sid * _KT
            pltpu.sync_copy(acc.at[pl.ds(pl.multiple_of(off, 256), _KT)],
                            h_ref.at[pl.ds(pl.multiple_of(base + off, 256), _KT)])
            pltpu.sync_copy(zer_ref, acc.at[pl.ds(pl.multiple_of(off, 256), _KT)])
            plsc.subcore_barrier()
            return 0
        lax.fori_loop(0, _NPASS, _pass, 0)

    return build(edge_flat, zer_b)


def _encode(x, H32, dcol, bcol, W1, b1, g1, bt1, W2, b2, g2, bt2, W3, b3):
    H = None
    h = x
    for (W, b, g, bt) in ((W1, b1, None, None),
                          (W2, b2, g1, bt1),
                          (W3, b3, g2, bt2)):
        if g is None:
            xw = _mm_plain(h, W)
            t, H = _ht_mm_conv(H32, xw, bcol)
        else:
            stats = _bn_stats(h)
            xw = _mm_bn(h, stats, g.reshape(1, -1), bt.reshape(1, -1), W)
            t = _ht_mm(H, xw, bcol)
        h = _h_mm(H, t, dcol, b.reshape(1, -1))
    return h


def kernel(x, edge, W1, b1, g1, bt1, W2, b2, g2, bt2, W3, b3):
    H_flat, D, B = _build_incidence_sc(edge)
    return _encode(x, H_flat.reshape(_N, _HEP), D[:_N].reshape(_N, 1),
                   B.reshape(_HEP, 1), W1, b1, g1, bt1, W2, b2, g2, bt2, W3, b3)


# speedup vs baseline: 1.0098x; 1.0000x over previous
import functools

import jax
import jax.numpy as jnp
from jax import lax
from jax.experimental import pallas as pl
from jax.experimental.pallas import tpu as pltpu
from jax.experimental.pallas import tpu_sc as plsc

N_HE = 10000
EPS = 1e-5
_INTERP = False


def _pick(b, n):
    return b if n % b == 0 else n


def _mm_plain(act, W):
    n, fin = act.shape
    fout = W.shape[1]
    rb = _pick(1000, n)

    def body(a_ref, w_ref, o_ref):
        acc = jnp.dot(a_ref[...], w_ref[...], preferred_element_type=jnp.float32)
        hi = acc.astype(jnp.bfloat16)
        lo = (acc - hi.astype(jnp.float32)).astype(jnp.bfloat16)
        o_ref[...] = jnp.concatenate([hi, lo], axis=1)

    return pl.pallas_call(
        body,
        grid=(n // rb,),
        in_specs=[pl.BlockSpec((rb, fin), lambda i: (i, 0)),
                  pl.BlockSpec((fin, fout), lambda i: (0, 0))],
        out_specs=pl.BlockSpec((rb, 2 * fout), lambda i: (i, 0)),
        out_shape=jax.ShapeDtypeStruct((n, 2 * fout), jnp.bfloat16),
        interpret=_INTERP,
    )(act, W)


def _mm_bn(h, stats, g2d, bt2d, W):
    n, fin = h.shape
    fout = W.shape[1]
    rb = _pick(1000, n)
    inv_n = 1.0 / n

    def body(h_ref, s_ref, g_ref, b_ref, w_ref, o_ref):
        mean = s_ref[0:1, :] * inv_n
        var = s_ref[1:2, :] * inv_n - mean * mean
        scale = g_ref[...] * lax.rsqrt(var + EPS)
        shift = b_ref[...] - mean * scale
        a = h_ref[...] * scale + shift
        acc = jnp.dot(a, w_ref[...], preferred_element_type=jnp.float32)
        hi = acc.astype(jnp.bfloat16)
        lo = (acc - hi.astype(jnp.float32)).astype(jnp.bfloat16)
        o_ref[...] = jnp.concatenate([hi, lo], axis=1)

    return pl.pallas_call(
        body,
        grid=(n // rb,),
        in_specs=[pl.BlockSpec((rb, fin), lambda i: (i, 0)),
                  pl.BlockSpec((2, fin), lambda i: (0, 0)),
                  pl.BlockSpec((1, fin), lambda i: (0, 0)),
                  pl.BlockSpec((1, fin), lambda i: (0, 0)),
                  pl.BlockSpec((fin, fout), lambda i: (0, 0))],
        out_specs=pl.BlockSpec((rb, 2 * fout), lambda i: (i, 0)),
        out_shape=jax.ShapeDtypeStruct((n, 2 * fout), jnp.bfloat16),
        interpret=_INTERP,
    )(h, stats, g2d, bt2d, W)


def _ht_mm(H, xw, b_col):
    n, hep = H.shape
    f = xw.shape[1]
    cb = _pick(512, hep)

    def body(h_ref, x_ref, s_ref, o_ref):
        acc = lax.dot_general(h_ref[...], x_ref[...],
                              (((0,), (0,)), ((), ())),
                              preferred_element_type=jnp.float32)
        fo = acc.shape[1] // 2
        cnt = s_ref[...]
        inv = jnp.where(cnt > 0, 1.0 / cnt, 0.0)
        val = (acc[:, :fo] + acc[:, fo:]) * inv
        hi = val.astype(jnp.bfloat16)
        lo = (val - hi.astype(jnp.float32)).astype(jnp.bfloat16)
        o_ref[...] = jnp.concatenate([hi, lo], axis=1)

    return pl.pallas_call(
        body,
        grid=(hep // cb,),
        in_specs=[pl.BlockSpec((n, cb), lambda i: (0, i)),
                  pl.BlockSpec((n, f), lambda i: (0, 0)),
                  pl.BlockSpec((cb, 1), lambda i: (i, 0))],
        out_specs=pl.BlockSpec((cb, f), lambda i: (i, 0)),
        out_shape=jax.ShapeDtypeStruct((hep, f), jnp.bfloat16),
        interpret=_INTERP,
    )(H, xw, b_col)


def _ht_mm_conv(H32, xw, b_col):
    n, hep = H32.shape
    f = xw.shape[1]
    cb = _pick(256, hep)

    def body(h_ref, x_ref, s_ref, o_ref, hb_ref):
        hb = h_ref[...].astype(jnp.bfloat16)
        hb_ref[...] = hb
        acc = lax.dot_general(hb, x_ref[...],
                              (((0,), (0,)), ((), ())),
                              preferred_element_type=jnp.float32)
        fo = acc.shape[1] // 2
        cnt = s_ref[...]
        inv = jnp.where(cnt > 0, 1.0 / cnt, 0.0)
        val = (acc[:, :fo] + acc[:, fo:]) * inv
        hi = val.astype(jnp.bfloat16)
        lo = (val - hi.astype(jnp.float32)).astype(jnp.bfloat16)
        o_ref[...] = jnp.concatenate([hi, lo], axis=1)

    return pl.pallas_call(
        body,
        grid=(hep // cb,),
        in_specs=[pl.BlockSpec((n, cb), lambda i: (0, i)),
                  pl.BlockSpec((n, f), lambda i: (0, 0)),
                  pl.BlockSpec((cb, 1), lambda i: (i, 0))],
        out_specs=[pl.BlockSpec((cb, f), lambda i: (i, 0)),
                   pl.BlockSpec((n, cb), lambda i: (0, i))],
        out_shape=[jax.ShapeDtypeStruct((hep, f), jnp.bfloat16),
                   jax.ShapeDtypeStruct((n, hep), jnp.bfloat16)],
        interpret=_INTERP,
    )(H32, xw, b_col)


def _h_mm(H, t, d_col, b2d):
    n, hep = H.shape
    f = t.shape[1]
    fo = f // 2
    rb = _pick(400, n)

    def body(h_ref, t_ref, s_ref, b_ref, o_ref):
        acc = jnp.dot(h_ref[...], t_ref[...], preferred_element_type=jnp.float32)
        fo = acc.shape[1] // 2
        val = acc[:, :fo] + acc[:, fo:]
        cnt = s_ref[...]
        inv = jnp.where(cnt > 0, 1.0 / cnt, 0.0)
        o_ref[...] = jnp.maximum(val * inv + b_ref[...], 0.0)

    return pl.pallas_call(
        body,
        grid=(n // rb,),
        in_specs=[pl.BlockSpec((rb, hep), lambda i: (i, 0)),
                  pl.BlockSpec((hep, f), lambda i: (0, 0)),
                  pl.BlockSpec((rb, 1), lambda i: (i, 0)),
                  pl.BlockSpec((1, fo), lambda i: (0, 0))],
        out_specs=pl.BlockSpec((rb, fo), lambda i: (i, 0)),
        out_shape=jax.ShapeDtypeStruct((n, fo), jnp.float32),
        interpret=_INTERP,
    )(H, t, d_col, b2d)


def _bn_stats(h):
    n, f = h.shape
    rb = _pick(1000, n)

    def body(h_ref, o_ref):
        i = pl.program_id(0)

        @pl.when(i == 0)
        def _():
            o_ref[...] = jnp.zeros_like(o_ref)

        v = h_ref[...]
        s = jnp.sum(v, axis=0, keepdims=True)
        ss = jnp.sum(v * v, axis=0, keepdims=True)
        o_ref[...] += jnp.concatenate([s, ss], axis=0)

    return pl.pallas_call(
        body,
        grid=(n // rb,),
        in_specs=[pl.BlockSpec((rb, f), lambda i: (i, 0))],
        out_specs=pl.BlockSpec((2, f), lambda i: (0, 0)),
        out_shape=jax.ShapeDtypeStruct((2, f), jnp.float32),
        interpret=_INTERP,
    )(h)


_N = 10000
_HEP = 10240
_E = 320000
_EPT = 10240
_EP = 32 * _EPT
_K = 1_024_000
_KT = _K // 16
_TOTAL = _N * _HEP
_NPASS = _TOTAL // (2 * _K)
_SENT = 200000


def _build_incidence_sc(edge):
    mesh = plsc.VectorSubcoreMesh(core_axis_name="c", subcore_axis_name="s")
    pad = jnp.full((2, _EP - _E), _SENT, jnp.int32)
    edge_flat = jnp.concatenate([edge, pad], axis=1).reshape(2 * _EP)
    zer_b = jnp.zeros((_KT,), jnp.float32)

    @functools.partial(
        pl.kernel,
        out_type=(
            jax.ShapeDtypeStruct((_TOTAL,), jnp.float32),
            jax.ShapeDtypeStruct((_HEP,), jnp.float32),
            jax.ShapeDtypeStruct((_HEP,), jnp.float32),
        ),
        mesh=mesh,
        scratch_types=(
            pltpu.VMEM((_EPT,), jnp.int32),
            pltpu.VMEM((_EPT,), jnp.int32),
            pltpu.VMEM((_EPT,), jnp.int32),
            pltpu.VMEM((_EPT,), jnp.int32),
            pltpu.VMEM((_EPT,), jnp.int32),
            pltpu.VMEM((_EPT,), jnp.float32),
            pltpu.VMEM_SHARED((_K + 1024,), jnp.float32),
            pltpu.VMEM_SHARED((_HEP + 1024,), jnp.float32),
        ),
    )
    def build(edge_ref, zer_ref, h_ref, d_ref, b_ref,
              n1, h1, n2, h2, idx1, vals1, acc, accd):
        cid = lax.axis_index("c")
        sid = lax.axis_index("s")

        onef = jnp.ones((16,), jnp.float32)
        lax.fori_loop(0, _EPT // 16,
                      lambda i, _: (vals1.__setitem__(pl.ds(i * 16, 16), onef), 0)[1],
                      0, unroll=8)

        pltpu.sync_copy(edge_ref.at[pl.ds(pl.multiple_of(sid * _EPT, 256), _EPT)],
                        n1)
        pltpu.sync_copy(edge_ref.at[pl.ds(pl.multiple_of(_EP + sid * _EPT, 256), _EPT)],
                        h1)
        pltpu.sync_copy(edge_ref.at[pl.ds(pl.multiple_of((16 + sid) * _EPT, 256), _EPT)],
                        n2)
        pltpu.sync_copy(edge_ref.at[pl.ds(pl.multiple_of(_EP + (16 + sid) * _EPT, 256), _EPT)],
                        h2)

        pltpu.sync_copy(zer_ref, acc.at[pl.ds(pl.multiple_of(sid * _KT, 256), _KT)])

        @pl.when(sid == 0)
        def _():
            pltpu.sync_copy(zer_ref.at[pl.ds(0, 1024)], acc.at[pl.ds(_K, 1024)])
            pltpu.sync_copy(zer_ref.at[pl.ds(0, _HEP + 1024)], accd)
        plsc.subcore_barrier()

        def fill_idx(src, base, limit, trash):
            def _row(r, _):
                for c in range(8):
                    s = r * 128 + c * 16
                    v = src[pl.ds(s, 16)]
                    off = v - base
                    ok = (off >= 0) & (off < limit)
                    idx1[pl.ds(s, 16)] = jnp.where(ok, off, trash + (v & 1023))
                return 0
            lax.fori_loop(0, 80, _row, 0)

        def scatter_add(dst):
            pltpu.sync_copy(vals1, dst.at[idx1], add=True)

        @pl.when(cid == 0)
        def _():
            fill_idx(n1, 0, _HEP, _HEP)
        @pl.when(cid == 1)
        def _():
            fill_idx(h1, 0, _HEP, _HEP)
        scatter_add(accd)
        @pl.when(cid == 0)
        def _():
            fill_idx(n2, 0, _HEP, _HEP)
        @pl.when(cid == 1)
        def _():
            fill_idx(h2, 0, _HEP, _HEP)
        scatter_add(accd)
        plsc.subcore_barrier()

        @pl.when((sid == 0) & (cid == 0))
        def _():
            pltpu.sync_copy(accd.at[pl.ds(0, _HEP)], d_ref)

        @pl.when((sid == 0) & (cid == 1))
        def _():
            pltpu.sync_copy(accd.at[pl.ds(0, _HEP)], b_ref)

        def _keys(r, _):
            for c in range(8):
                s = r * 128 + c * 16
                n1[pl.ds(s, 16)] = n1[pl.ds(s, 16)] * _HEP + h1[pl.ds(s, 16)]
                n2[pl.ds(s, 16)] = n2[pl.ds(s, 16)] * _HEP + h2[pl.ds(s, 16)]
            return 0
        lax.fori_loop(0, 80, _keys, 0)

        def _pass(p, _):
            base = (2 * p + cid) * _K
            fill_idx(n1, base, _K, _K)
            scatter_add(acc)
            fill_idx(n2, base, _K, _K)
            scatter_add(acc)
            plsc.subcore_barrier()
            off = sid * _KT
            pltpu.sync_copy(acc.at[pl.ds(pl.multiple_of(off, 256), _KT)],
                            h_ref.at[pl.ds(pl.multiple_of(base + off, 256), _KT)])
            pltpu.sync_copy(zer_ref, acc.at[pl.ds(pl.multiple_of(off, 256), _KT)])
            plsc.subcore_barrier()
            return 0
        lax.fori_loop(0, _NPASS, _pass, 0)

    return build(edge_flat, zer_b)


def _encode(x, H32, dcol, bcol, W1, b1, g1, bt1, W2, b2, g2, bt2, W3, b3):
    H = None
    h = x
    for (W, b, g, bt) in ((W1, b1, None, None),
                          (W2, b2, g1, bt1),
                          (W3, b3, g2, bt2)):
        if g is None:
            xw = _mm_plain(h, W)
            t, H = _ht_mm_conv(H32, xw, bcol)
        else:
            stats = _bn_stats(h)
            xw = _mm_bn(h, stats, g.reshape(1, -1), bt.reshape(1, -1), W)
            t = _ht_mm(H, xw, bcol)
        h = _h_mm(H, t, dcol, b.reshape(1, -1))
    return h


def kernel(x, edge, W1, b1, g1, bt1, W2, b2, g2, bt2, W3, b3):
    H_flat, D, B = _build_incidence_sc(edge)
    return _encode(x, H_flat.reshape(_N, _HEP), D[:_N].reshape(_N, 1),
                   B.reshape(_HEP, 1), W1, b1, g1, bt1, W2, b2, g2, bt2, W3, b3)
